# baseline (speedup 1.0000x reference)
import functools

import jax
import jax.numpy as jnp
from jax import lax
from jax.experimental import pallas as pl
from jax.experimental.pallas import tpu as pltpu
from jax.experimental.pallas import tpu_sc as plsc

N = 10000
E = 320000
D = 128
DH = D // 2
H = 128
NPAD = 10240
NC, NS = 2, 16
ET = E // NS
CHUNK = 80
NCH = ET // CHUNK
RPS = NPAD // NS
AW = DH + 16


def _prep_body(x_ref, w_ref, asrc_w_ref, adst_w_ref, xw2_ref, asrc_ref,
               adst_ref):
    xw = jnp.dot(x_ref[...], w_ref[...], preferred_element_type=jnp.float32)
    xw2_ref[0] = xw[:, :DH]
    xw2_ref[1] = xw[:, DH:]
    asrc_ref[...] = jnp.sum(xw * asrc_w_ref[...], axis=1, keepdims=True)
    adst_ref[...] = jnp.sum(xw * adst_w_ref[...], axis=1, keepdims=True)


def _prep(x, W, att_src, att_dst):
    R = 2000
    return pl.pallas_call(
        _prep_body,
        grid=(N // R,),
        in_specs=[
            pl.BlockSpec((R, D), lambda i: (i, 0)),
            pl.BlockSpec((D, D), lambda i: (0, 0)),
            pl.BlockSpec((1, D), lambda i: (0, 0)),
            pl.BlockSpec((1, D), lambda i: (0, 0)),
        ],
        out_specs=[
            pl.BlockSpec((NC, R, DH), lambda i: (0, i, 0)),
            pl.BlockSpec((R, 1), lambda i: (i, 0)),
            pl.BlockSpec((R, 1), lambda i: (i, 0)),
        ],
        out_shape=[
            jax.ShapeDtypeStruct((NC, N, DH), jnp.float32),
            jax.ShapeDtypeStruct((N, 1), jnp.float32),
            jax.ShapeDtypeStruct((N, 1), jnp.float32),
        ],
    )(x, W, att_src.reshape(1, D), att_dst.reshape(1, D))


@functools.partial(
    pl.kernel,
    out_type=jax.ShapeDtypeStruct((NC, NPAD, AW), jnp.float32),
    mesh=plsc.VectorSubcoreMesh(core_axis_name="c", subcore_axis_name="s"),
    compiler_params=pltpu.CompilerParams(needs_layout_passes=False,
                                         use_tc_tiling_on_sc=False),
    scratch_types=[
        pltpu.VMEM((ET,), jnp.int32),
        pltpu.VMEM((ET,), jnp.int32),
        pltpu.VMEM((CHUNK,), jnp.int32),
        pltpu.VMEM((N,), jnp.float32),
        pltpu.VMEM((N,), jnp.float32),
        pltpu.VMEM((2, CHUNK, DH), jnp.float32),
        pltpu.VMEM((CHUNK, AW), jnp.float32),
        pltpu.VMEM_SHARED((NPAD, AW), jnp.float32),
        pltpu.SemaphoreType.DMA((2,)),
    ],
)
def _sc_edge(xw2_hbm, src_hbm, dst_hbm, asrc_hbm, adst_hbm,
             num_hbm,
             src_v, dst_v, dstc_v, asrc_v, adst_v, rowsg_v, rows_v, num_sh,
             gsem):
    c = lax.axis_index("c")
    s = lax.axis_index("s")
    zeros16 = jnp.zeros((16,), jnp.float32)
    wcol16 = jnp.full((16,), DH, jnp.int32)
    iota16 = lax.iota(jnp.int32, 16)

    def _zrows(i, carry):
        for r in range(AW // 16):
            rows_v[i, pl.ds(r * 16, 16)] = zeros16
        return carry
    lax.fori_loop(0, CHUNK, _zrows, 0)
    for t in range(RPS // CHUNK):
        pltpu.sync_copy(rows_v,
                        num_sh.at[pl.ds(s * RPS + t * CHUNK, CHUNK)])

    pltpu.sync_copy(asrc_hbm, asrc_v)
    pltpu.sync_copy(adst_hbm, adst_v)
    ebase = pl.multiple_of(s * ET, 8)
    pltpu.sync_copy(src_hbm.at[pl.ds(ebase, ET)], src_v)
    pltpu.sync_copy(dst_hbm.at[pl.ds(ebase, ET)], dst_v)
    plsc.subcore_barrier()

    def _gather(k, b):
        off = pl.multiple_of(k * CHUNK, 8)
        return pltpu.async_copy(
            xw2_hbm.at[c].at[src_v.at[pl.ds(off, CHUNK)]],
            rowsg_v.at[b], gsem.at[b])

    def _gather_wait(k, b):
        off = pl.multiple_of(k * CHUNK, 8)
        pltpu.make_async_copy(
            xw2_hbm.at[c].at[src_v.at[pl.ds(off, CHUNK)]],
            rowsg_v.at[b], gsem.at[b]).wait()

    _gather(0, 0)

    def _pair(kk, carry):
        for b in range(2):
            k = kk * 2 + b
            off = pl.multiple_of(k * CHUNK, 8)

            @pl.when(k + 1 < NCH)
            def _():
                _gather(k + 1, 1 - b)
            _gather_wait(k, b)

            for g in range(CHUNK // 16):
                sv = src_v[pl.ds(off + g * 16, 16)]
                dv = dst_v[pl.ds(off + g * 16, 16)]
                dstc_v[pl.ds(g * 16, 16)] = dv
                e = (plsc.load_gather(asrc_v, [sv])
                     + plsc.load_gather(adst_v, [dv]))
                e = jnp.where(e >= 0.0, e, 0.2 * e)
                w16 = jnp.exp(e)
                plsc.store_scatter(rows_v, [iota16 + (g * 16), wcol16], w16)
                for j in range(16):
                    wj = w16[j]
                    row = g * 16 + j
                    for r in range(DH // 16):
                        rows_v[row, pl.ds(r * 16, 16)] = (
                            rowsg_v[b, row, pl.ds(r * 16, 16)] * wj)

            pltpu.sync_copy(rows_v, num_sh.at[dstc_v], add=True)
        return carry
    lax.fori_loop(0, NCH // 2, _pair, 0)

    plsc.subcore_barrier()
    pltpu.sync_copy(num_sh.at[pl.ds(s * RPS, RPS)],
                    num_hbm.at[c, pl.ds(s * RPS, RPS)])


def _final_body(num_ref, b_ref, h_ref, c_ref, wih_ref, whh_ref,
                h1_ref, c1_ref):
    nsum = jnp.concatenate([num_ref[0, :, :DH], num_ref[1, :, :DH]], axis=1)
    dsum = num_ref[0, :, DH]
    xb = jnp.tanh(nsum / (dsum[:, None] + 1e-16) + b_ref[...])
    dn = (((1,), (1,)), ((), ()))
    gates = lax.dot_general(xb, wih_ref[...], dn,
                            preferred_element_type=jnp.float32)
    gates = gates + lax.dot_general(h_ref[...], whh_ref[...], dn,
                                    preferred_element_type=jnp.float32)
    i = jax.nn.sigmoid(gates[:, :H])
    f = jax.nn.sigmoid(gates[:, H:2 * H])
    g = jnp.tanh(gates[:, 2 * H:3 * H])
    o = jax.nn.sigmoid(gates[:, 3 * H:])
    c1 = f * c_ref[...] + i * g
    h1_ref[...] = o * jnp.tanh(c1)
    c1_ref[...] = c1


def _final(num, b, h0, c0, wihT, whhT):
    R = 2000
    return pl.pallas_call(
        _final_body,
        grid=(N // R,),
        in_specs=[
            pl.BlockSpec((NC, R, AW), lambda i: (0, i, 0)),
            pl.BlockSpec((1, D), lambda i: (0, 0)),
            pl.BlockSpec((R, H), lambda i: (i, 0)),
            pl.BlockSpec((R, H), lambda i: (i, 0)),
            pl.BlockSpec((4 * H, D), lambda i: (0, 0)),
            pl.BlockSpec((4 * H, H), lambda i: (0, 0)),
        ],
        out_specs=[
            pl.BlockSpec((R, H), lambda i: (i, 0)),
            pl.BlockSpec((R, H), lambda i: (i, 0)),
        ],
        out_shape=[
            jax.ShapeDtypeStruct((N, H), jnp.float32),
            jax.ShapeDtypeStruct((N, H), jnp.float32),
        ],
    )(num, b, h0, c0, wihT, whhT)


def kernel(x, edge_index, h, c, W_gat, att_src, att_dst, bias_gat, W_ih, W_hh):
    src = edge_index[0].astype(jnp.int32)
    dst = edge_index[1].astype(jnp.int32)
    xw2, asrc, adst = _prep(x, W_gat, att_src, att_dst)
    num = _sc_edge(xw2, src, dst, asrc.reshape(N), adst.reshape(N))
    h1, c1 = _final(num, bias_gat.reshape(1, D),
                    h[0], c[0], W_ih, W_hh)
    return (h1, h1[None, :, :], c1[None, :, :])


# speedup vs baseline: 1.0946x; 1.0946x over previous
import functools

import jax
import jax.numpy as jnp
from jax import lax
from jax.experimental import pallas as pl
from jax.experimental.pallas import tpu as pltpu
from jax.experimental.pallas import tpu_sc as plsc

N = 10000
E = 320000
D = 128
DH = D // 2
H = 128
NPAD = 10240
NC, NS = 2, 16
ET = E // NS
CHUNK = 80
NCH = ET // CHUNK
RPS = NPAD // NS
AW = DH + 16


def _prep_body(x_ref, w_ref, asrc_w_ref, adst_w_ref, xw2_ref, asrc_ref,
               adst_ref):
    xw = jnp.dot(x_ref[...], w_ref[...], preferred_element_type=jnp.float32)
    xw2_ref[0] = xw[:, :DH].astype(jnp.bfloat16)
    xw2_ref[1] = xw[:, DH:].astype(jnp.bfloat16)
    asrc_ref[...] = jnp.sum(xw * asrc_w_ref[...], axis=1, keepdims=True)
    adst_ref[...] = jnp.sum(xw * adst_w_ref[...], axis=1, keepdims=True)


def _prep(x, W, att_src, att_dst):
    R = 2000
    return pl.pallas_call(
        _prep_body,
        grid=(N // R,),
        in_specs=[
            pl.BlockSpec((R, D), lambda i: (i, 0)),
            pl.BlockSpec((D, D), lambda i: (0, 0)),
            pl.BlockSpec((1, D), lambda i: (0, 0)),
            pl.BlockSpec((1, D), lambda i: (0, 0)),
        ],
        out_specs=[
            pl.BlockSpec((NC, R, DH), lambda i: (0, i, 0)),
            pl.BlockSpec((R, 1), lambda i: (i, 0)),
            pl.BlockSpec((R, 1), lambda i: (i, 0)),
        ],
        out_shape=[
            jax.ShapeDtypeStruct((NC, N, DH), jnp.bfloat16),
            jax.ShapeDtypeStruct((N, 1), jnp.float32),
            jax.ShapeDtypeStruct((N, 1), jnp.float32),
        ],
    )(x, W, att_src.reshape(1, D), att_dst.reshape(1, D))


@functools.partial(
    pl.kernel,
    out_type=jax.ShapeDtypeStruct((NC, NPAD, AW), jnp.float32),
    mesh=plsc.VectorSubcoreMesh(core_axis_name="c", subcore_axis_name="s"),
    compiler_params=pltpu.CompilerParams(needs_layout_passes=False,
                                         use_tc_tiling_on_sc=False),
    scratch_types=[
        pltpu.VMEM((ET,), jnp.int32),
        pltpu.VMEM((ET,), jnp.int32),
        pltpu.VMEM((2, CHUNK), jnp.int32),
        pltpu.VMEM((N,), jnp.float32),
        pltpu.VMEM((N,), jnp.float32),
        pltpu.VMEM((2, CHUNK, DH), jnp.bfloat16),
        pltpu.VMEM((2, CHUNK, AW), jnp.float32),
        pltpu.VMEM_SHARED((NPAD, AW), jnp.float32),
        pltpu.SemaphoreType.DMA((2,)),
        pltpu.SemaphoreType.DMA((2,)),
    ],
)
def _sc_edge(xw2_hbm, src_hbm, dst_hbm, asrc_hbm, adst_hbm,
             num_hbm,
             src_v, dst_v, dstc_v, asrc_v, adst_v, rowsg_v, rows_v, num_sh,
             gsem, ssem):
    c = lax.axis_index("c")
    s = lax.axis_index("s")
    zeros16 = jnp.zeros((16,), jnp.float32)
    wcol16 = jnp.full((16,), DH, jnp.int32)
    iota16 = lax.iota(jnp.int32, 16)

    def _zrows(i, carry):
        for b in range(2):
            for r in range(AW // 16):
                rows_v[b, i, pl.ds(r * 16, 16)] = zeros16
        return carry
    lax.fori_loop(0, CHUNK, _zrows, 0)
    for t in range(RPS // CHUNK):
        pltpu.sync_copy(rows_v.at[0],
                        num_sh.at[pl.ds(s * RPS + t * CHUNK, CHUNK)])

    pltpu.sync_copy(asrc_hbm, asrc_v)
    pltpu.sync_copy(adst_hbm, adst_v)
    ebase = pl.multiple_of(s * ET, 8)
    pltpu.sync_copy(src_hbm.at[pl.ds(ebase, ET)], src_v)
    pltpu.sync_copy(dst_hbm.at[pl.ds(ebase, ET)], dst_v)
    plsc.subcore_barrier()

    def _gather(k, b):
        off = pl.multiple_of(k * CHUNK, 8)
        return pltpu.async_copy(
            xw2_hbm.at[c].at[src_v.at[pl.ds(off, CHUNK)]],
            rowsg_v.at[b], gsem.at[b])

    def _gather_wait(k, b):
        off = pl.multiple_of(k * CHUNK, 8)
        pltpu.make_async_copy(
            xw2_hbm.at[c].at[src_v.at[pl.ds(off, CHUNK)]],
            rowsg_v.at[b], gsem.at[b]).wait()

    def _scatter_wait(b):
        pltpu.make_async_copy(rows_v.at[b], num_sh.at[dstc_v.at[b]],
                              ssem.at[b]).wait()

    _gather(0, 0)

    def _pair(kk, carry):
        for b in range(2):
            k = kk * 2 + b
            off = pl.multiple_of(k * CHUNK, 8)

            @pl.when(k + 1 < NCH)
            def _():
                _gather(k + 1, 1 - b)
            _gather_wait(k, b)

            @pl.when(k >= 2)
            def _():
                _scatter_wait(b)

            for g in range(CHUNK // 16):
                sv = src_v[pl.ds(off + g * 16, 16)]
                dv = dst_v[pl.ds(off + g * 16, 16)]
                dstc_v[b, pl.ds(g * 16, 16)] = dv
                e = (plsc.load_gather(asrc_v, [sv])
                     + plsc.load_gather(adst_v, [dv]))
                e = jnp.where(e >= 0.0, e, 0.2 * e)
                w16 = jnp.exp(e)
                plsc.store_scatter(rows_v.at[b], [iota16 + (g * 16), wcol16],
                                   w16)
                for j in range(16):
                    wj = w16[j]
                    row = g * 16 + j
                    for hf in range(2):
                        xi = plsc.bitcast(
                            rowsg_v[b, row, pl.ds(hf * 32, 32)], jnp.int32)
                        fe = plsc.bitcast(xi << 16, jnp.float32) * wj
                        fo = plsc.bitcast(
                            (xi >> 16) << 16, jnp.float32) * wj
                        rows_v[b, row, pl.ds(hf * 32, 16)] = fe
                        rows_v[b, row, pl.ds(hf * 32 + 16, 16)] = fo

            pltpu.async_copy(rows_v.at[b], num_sh.at[dstc_v.at[b]],
                             ssem.at[b], add=True)
        return carry
    lax.fori_loop(0, NCH // 2, _pair, 0)
    for b in range(2):
        _scatter_wait(b)

    plsc.subcore_barrier()
    pltpu.sync_copy(num_sh.at[pl.ds(s * RPS, RPS)],
                    num_hbm.at[c, pl.ds(s * RPS, RPS)])


def _final_body(num_ref, b_ref, h_ref, c_ref, wih_ref, whh_ref,
                h1_ref, c1_ref):
    nsum = jnp.concatenate([num_ref[0, :, :DH], num_ref[1, :, :DH]], axis=1)
    dsum = num_ref[0, :, DH]
    xb = jnp.tanh(nsum / (dsum[:, None] + 1e-16) + b_ref[...])
    dn = (((1,), (1,)), ((), ()))
    gates = lax.dot_general(xb, wih_ref[...], dn,
                            preferred_element_type=jnp.float32)
    gates = gates + lax.dot_general(h_ref[...], whh_ref[...], dn,
                                    preferred_element_type=jnp.float32)
    i = jax.nn.sigmoid(gates[:, :H])
    f = jax.nn.sigmoid(gates[:, H:2 * H])
    g = jnp.tanh(gates[:, 2 * H:3 * H])
    o = jax.nn.sigmoid(gates[:, 3 * H:])
    c1 = f * c_ref[...] + i * g
    h1_ref[...] = o * jnp.tanh(c1)
    c1_ref[...] = c1


def _final(num, b, h0, c0, wihT, whhT):
    R = 2000
    return pl.pallas_call(
        _final_body,
        grid=(N // R,),
        in_specs=[
            pl.BlockSpec((NC, R, AW), lambda i: (0, i, 0)),
            pl.BlockSpec((1, D), lambda i: (0, 0)),
            pl.BlockSpec((R, H), lambda i: (i, 0)),
            pl.BlockSpec((R, H), lambda i: (i, 0)),
            pl.BlockSpec((4 * H, D), lambda i: (0, 0)),
            pl.BlockSpec((4 * H, H), lambda i: (0, 0)),
        ],
        out_specs=[
            pl.BlockSpec((R, H), lambda i: (i, 0)),
            pl.BlockSpec((R, H), lambda i: (i, 0)),
        ],
        out_shape=[
            jax.ShapeDtypeStruct((N, H), jnp.float32),
            jax.ShapeDtypeStruct((N, H), jnp.float32),
        ],
    )(num, b, h0, c0, wihT, whhT)


def kernel(x, edge_index, h, c, W_gat, att_src, att_dst, bias_gat, W_ih, W_hh):
    src = edge_index[0].astype(jnp.int32)
    dst = edge_index[1].astype(jnp.int32)
    xw2, asrc, adst = _prep(x, W_gat, att_src, att_dst)
    perm = jnp.array([hf * 32 + (t // 2) + 16 * (t % 2)
                      for hf in range(2) for t in range(32)], jnp.int32)
    xw2 = xw2[:, :, perm]
    num = _sc_edge(xw2, src, dst, asrc.reshape(N), adst.reshape(N))
    h1, c1 = _final(num, bias_gat.reshape(1, D),
                    h[0], c[0], W_ih, W_hh)
    return (h1, h1[None, :, :], c1[None, :, :])
